# Initial kernel scaffold; baseline (speedup 1.0000x reference)
import jax
import jax.numpy as jnp
from jax.experimental import pallas as pl


def kernel(pred_bboxes, pred_scores, priors, gt_labels, gt_bboxes, pad_bbox_flag):
    raise NotImplementedError("write your pallas kernel here")


# speedup vs baseline: 39.5035x; 39.5035x over previous
import jax
import jax.numpy as jnp
from jax import lax
from jax.experimental import pallas as pl

NUM_CLASSES = 80
SOFT_CENTER_RADIUS = 3.0
TOPK = 13
IOU_WEIGHT = 3.0
INF = 100000000.0
EPS = 1e-7
BIG = 3.0e38
LN10 = 2.302585092994046


def _assigner_kernel(pb_ref, ps_ref, pr_ref, gt_ref, lab_ref, flag_ref,
                     lab_out, bbox_out, met_out):
    N = pb_ref.shape[2]
    G = gt_ref.shape[1]

    pb = pb_ref[0]
    pr = pr_ref[...]
    gt = gt_ref[0]
    lab = lab_ref[0]
    flag = flag_ref[0]

    px = pr[0:1, :]
    py = pr[1:2, :]
    pstride = pr[2:3, :]
    x1 = pb[0:1, :]
    y1 = pb[1:2, :]
    x2 = pb[2:3, :]
    y2 = pb[3:4, :]
    gx1 = gt[:, 0:1]
    gy1 = gt[:, 1:2]
    gx2 = gt[:, 2:3]
    gy2 = gt[:, 3:4]

    in_gts = (px > gx1) & (py > gy1) & (px < gx2) & (py < gy2) & (flag > 0)
    valid = jnp.sum(in_gts.astype(jnp.float32), axis=0, keepdims=True) > 0
    validf = valid.astype(jnp.float32)

    gcx = (gx1 + gx2) * 0.5
    gcy = (gy1 + gy2) * 0.5
    dist = jnp.sqrt((px - gcx) ** 2 + (py - gcy) ** 2) / pstride
    dist = dist * validf
    soft = jnp.power(10.0, dist - SOFT_CENTER_RADIUS)

    iw = jnp.maximum(jnp.minimum(x2, gx2) - jnp.maximum(x1, gx1), 0.0)
    ih = jnp.maximum(jnp.minimum(y2, gy2) - jnp.maximum(y1, gy1), 0.0)
    overlap = iw * ih
    area1 = (x2 - x1) * (y2 - y1)
    area2 = (gx2 - gx1) * (gy2 - gy1)
    union = jnp.maximum(area1 + area2 - overlap, 1e-6)
    iou = overlap / union
    iou_cost = -jnp.log(iou + EPS) * IOU_WEIGHT

    sc = ps_ref[0]
    cls_iota = lax.broadcasted_iota(jnp.int32, (G, NUM_CLASSES), 1)
    oh = (cls_iota == lab).astype(jnp.float32)
    x = jnp.dot(oh, sc, preferred_element_type=jnp.float32)
    sig = jax.nn.sigmoid(x)
    bce = jnp.maximum(x, 0.0) - x * iou + jnp.log1p(jnp.exp(-jnp.abs(x)))
    cost = bce * (iou - sig) ** 2 + iou_cost + soft
    cost = jnp.where(valid, cost, INF)

    pidx = lax.broadcasted_iota(jnp.int32, (G, N), 1)

    def iou_body(j, c):
        w, s = c
        m = jnp.max(w, axis=1, keepdims=True)
        fid = jnp.min(jnp.where(w == m, pidx, N), axis=1, keepdims=True)
        w = jnp.where(pidx == fid, -1.0, w)
        return (w, s + m)

    _, s13 = lax.fori_loop(0, TOPK, iou_body,
                           (iou, jnp.zeros((G, 1), jnp.float32)))
    ks = jnp.maximum(s13.astype(jnp.int32), 1)

    gt_ok = flag > 0

    def cost_body(j, c):
        w, mt = c
        m = jnp.min(w, axis=1, keepdims=True)
        fid = jnp.min(jnp.where(w == m, pidx, N), axis=1, keepdims=True)
        hit = pidx == fid
        sel = hit & (j < ks) & gt_ok
        w = jnp.where(hit, BIG, w)
        return (w, jnp.where(sel, 1.0, mt))

    _, matching = lax.fori_loop(0, TOPK, cost_body,
                                (cost, jnp.zeros((G, N), jnp.float32)))

    cnt = jnp.sum(matching, axis=0, keepdims=True)
    gidx = lax.broadcasted_iota(jnp.int32, (G, N), 0)
    rmin = jnp.min(cost, axis=0, keepdims=True)
    amin = jnp.min(jnp.where(cost == rmin, gidx, G), axis=0, keepdims=True)
    fmatch = jnp.min(jnp.where(matching > 0, gidx, G), axis=0, keepdims=True)
    mg = jnp.where(cnt > 1, amin, fmatch)
    fg = cnt > 0

    sel = (gidx == mg).astype(jnp.float32)
    met = jnp.sum(sel * iou, axis=0, keepdims=True)
    labf = jnp.sum(sel * lab.astype(jnp.float32), axis=0, keepdims=True)
    bbox = lax.dot_general(gt, sel, (((0,), (0,)), ((), ())),
                           preferred_element_type=jnp.float32)

    lab_out[0] = jnp.where(fg, labf.astype(jnp.int32), NUM_CLASSES)
    met_out[0] = jnp.where(fg, met, 0.0)
    bbox_out[0] = jnp.where(fg, bbox, 0.0)


def kernel(pred_bboxes, pred_scores, priors, gt_labels, gt_bboxes, pad_bbox_flag):
    B, N, _ = pred_bboxes.shape
    G = gt_bboxes.shape[1]
    pb_t = jnp.transpose(pred_bboxes, (0, 2, 1))
    ps_t = jnp.transpose(pred_scores, (0, 2, 1))
    pr_t = jnp.transpose(priors, (1, 0))
    lab = gt_labels.astype(jnp.int32)

    labels, bboxes_t, metrics = pl.pallas_call(
        _assigner_kernel,
        grid=(B,),
        in_specs=[
            pl.BlockSpec((1, 4, N), lambda b: (b, 0, 0)),
            pl.BlockSpec((1, NUM_CLASSES, N), lambda b: (b, 0, 0)),
            pl.BlockSpec((4, N), lambda b: (0, 0)),
            pl.BlockSpec((1, G, 4), lambda b: (b, 0, 0)),
            pl.BlockSpec((1, G, 1), lambda b: (b, 0, 0)),
            pl.BlockSpec((1, G, 1), lambda b: (b, 0, 0)),
        ],
        out_specs=[
            pl.BlockSpec((1, 1, N), lambda b: (b, 0, 0)),
            pl.BlockSpec((1, 4, N), lambda b: (b, 0, 0)),
            pl.BlockSpec((1, 1, N), lambda b: (b, 0, 0)),
        ],
        out_shape=[
            jax.ShapeDtypeStruct((B, 1, N), jnp.int32),
            jax.ShapeDtypeStruct((B, 4, N), jnp.float32),
            jax.ShapeDtypeStruct((B, 1, N), jnp.float32),
        ],
    )(pb_t, ps_t, pr_t, gt_bboxes, lab, pad_bbox_flag)

    weights = jnp.ones((B, N), dtype=gt_bboxes.dtype)
    bboxes = jnp.transpose(bboxes_t, (0, 2, 1))
    return labels[:, 0], weights, bboxes, metrics[:, 0]
